# baseline (speedup 1.0000x reference)
import functools

import jax
import jax.numpy as jnp
from jax import lax
from jax.experimental import pallas as pl
from jax.experimental.pallas import tpu as pltpu
from jax.experimental.pallas import tpu_sc as plsc

N = 10000
D = 128
E = 320000
NPAD = 10240
NW = 32
B = 128
CHUNKS = 80
IDXG = 8
NIG = CHUNKS // IDXG
EPT = CHUNKS * B
EPAD = EPT * NW
RPT = NPAD // 16
RB = 512
GRID = NPAD // RB

_mesh = plsc.VectorSubcoreMesh(core_axis_name="c", subcore_axis_name="s")


@functools.partial(
    pl.kernel,
    mesh=_mesh,
    out_type=jax.ShapeDtypeStruct((2 * NPAD, D), jnp.float32),
    scratch_types=[
        pltpu.VMEM((CHUNKS, B), jnp.int32),
        pltpu.VMEM((B, D), jnp.float32),
        pltpu.VMEM_SHARED((NPAD, D), jnp.float32),
    ],
)
def _deg_kernel(dst_hbm, ones_hbm, zeros_hbm, out_hbm, dst_v, ones_v, acc):
    cid = lax.axis_index("c")
    sid = lax.axis_index("s")
    wid = sid * 2 + cid
    pltpu.sync_copy(zeros_hbm, acc.at[pl.ds(sid * RPT, RPT)])
    pltpu.sync_copy(ones_hbm, ones_v)
    pltpu.sync_copy(dst_hbm.at[wid], dst_v)
    plsc.subcore_barrier()

    def body(j, c):
        pltpu.sync_copy(ones_v, acc.at[dst_v.at[j]], add=True)
        return c

    lax.fori_loop(0, CHUNKS, body, 0)
    plsc.subcore_barrier()
    pltpu.sync_copy(acc.at[pl.ds(sid * RPT, RPT)],
                    out_hbm.at[pl.ds(cid * NPAD + sid * RPT, RPT)])


@functools.partial(
    pl.kernel,
    mesh=_mesh,
    out_type=jax.ShapeDtypeStruct((2 * NPAD, D), jnp.float32),
    scratch_types=(
        [pltpu.VMEM((2, IDXG, B), jnp.int32),
         pltpu.VMEM((CHUNKS, B), jnp.int32)]
        + [pltpu.VMEM((B, D), jnp.float32)] * 2
        + [pltpu.VMEM_SHARED((NPAD, D), jnp.float32)]
        + [pltpu.SemaphoreType.DMA] * 5
    ),
)
def _scatter_kernel(y_hbm, src_hbm, dst_hbm, zeros_hbm, out_hbm,
                    src_v, dst_v, rows0, rows1, acc,
                    gsem0, gsem1, ssem0, ssem1, isem):
    rows = (rows0, rows1)
    gsem = (gsem0, gsem1)
    ssem = (ssem0, ssem1)
    cid = lax.axis_index("c")
    sid = lax.axis_index("s")
    wid = sid * 2 + cid
    pltpu.sync_copy(zeros_hbm, acc.at[pl.ds(sid * RPT, RPT)])
    pltpu.sync_copy(dst_hbm.at[wid], dst_v)

    def fire_idx(ig, p):
        pltpu.async_copy(src_hbm.at[wid * NIG + ig], src_v.at[p], isem)

    def wait_idx(ig, p):
        pltpu.make_async_copy(src_hbm.at[wid * NIG + ig], src_v.at[p],
                              isem).wait()

    fire_idx(0, 0)
    plsc.subcore_barrier()

    def group(ig, c):
        p = lax.rem(ig, 2)
        j0 = ig * IDXG

        def gather(jj, k):
            pltpu.async_copy(y_hbm.at[src_v.at[p, jj]], rows[k], gsem[k])

        def gather_wait(jj, k):
            pltpu.make_async_copy(y_hbm.at[src_v.at[p, jj]], rows[k],
                                  gsem[k]).wait()

        def scat(jj, k):
            pltpu.async_copy(rows[k], acc.at[dst_v.at[j0 + jj]], ssem[k],
                             add=True)

        def scat_wait(j, k):
            pltpu.make_async_copy(rows[k], acc.at[dst_v.at[j]],
                                  ssem[k]).wait()

        wait_idx(ig, p)

        @pl.when(ig + 1 < NIG)
        def _():
            fire_idx(ig + 1, 1 - p)

        for jj in range(IDXG):
            k = jj % 2
            gather(jj, k)
            gather_wait(jj, k)
            pltpu.sync_copy(rows[k], acc.at[dst_v.at[j0 + jj]], add=True)
        return c

    lax.fori_loop(0, NIG, group, 0)
    plsc.subcore_barrier()
    pltpu.sync_copy(acc.at[pl.ds(sid * RPT, RPT)],
                    out_hbm.at[pl.ds(cid * NPAD + sid * RPT, RPT)])


def _t1_body(x_ref, w_ref, degp_ref, y_ref, disb_ref):
    c = degp_ref[0, :, 0:1] + degp_ref[1, :, 0:1] + 1.0
    disb = lax.rsqrt(jnp.broadcast_to(c, (RB, D)))
    disb_ref[...] = disb
    y_ref[...] = jnp.dot(x_ref[...], w_ref[...],
                         preferred_element_type=jnp.float32) * disb


_t1 = pl.pallas_call(
    _t1_body,
    grid=(GRID,),
    in_specs=[
        pl.BlockSpec((RB, D), lambda i: (i, 0)),
        pl.BlockSpec((D, D), lambda i: (0, 0)),
        pl.BlockSpec((2, RB, D), lambda i: (0, i, 0)),
    ],
    out_specs=[
        pl.BlockSpec((RB, D), lambda i: (i, 0)),
        pl.BlockSpec((RB, D), lambda i: (i, 0)),
    ],
    out_shape=[
        jax.ShapeDtypeStruct((NPAD, D), jnp.float32),
        jax.ShapeDtypeStruct((NPAD, D), jnp.float32),
    ],
)


def _tmid_body(s_ref, y_ref, disb_ref, b_ref, w_ref, o_ref):
    disb = disb_ref[...]
    h = jnp.maximum((s_ref[0] + s_ref[1] + y_ref[...]) * disb + b_ref[...],
                    0.0)
    o_ref[...] = jnp.dot(h, w_ref[...],
                         preferred_element_type=jnp.float32) * disb


_tmid = pl.pallas_call(
    _tmid_body,
    grid=(GRID,),
    in_specs=[
        pl.BlockSpec((2, RB, D), lambda i: (0, i, 0)),
        pl.BlockSpec((RB, D), lambda i: (i, 0)),
        pl.BlockSpec((RB, D), lambda i: (i, 0)),
        pl.BlockSpec((1, D), lambda i: (0, 0)),
        pl.BlockSpec((D, D), lambda i: (0, 0)),
    ],
    out_specs=pl.BlockSpec((RB, D), lambda i: (i, 0)),
    out_shape=jax.ShapeDtypeStruct((NPAD, D), jnp.float32),
)


def _tfin_body(s_ref, y_ref, disb_ref, b_ref, o_ref):
    o_ref[...] = ((s_ref[0] + s_ref[1] + y_ref[...]) * disb_ref[...]
                  + b_ref[...])


_tfin = pl.pallas_call(
    _tfin_body,
    grid=(GRID,),
    in_specs=[
        pl.BlockSpec((2, RB, D), lambda i: (0, i, 0)),
        pl.BlockSpec((RB, D), lambda i: (i, 0)),
        pl.BlockSpec((RB, D), lambda i: (i, 0)),
        pl.BlockSpec((1, D), lambda i: (0, 0)),
    ],
    out_specs=pl.BlockSpec((RB, D), lambda i: (i, 0)),
    out_shape=jax.ShapeDtypeStruct((NPAD, D), jnp.float32),
)


def kernel(x, edge_index, W1, b1, W2, b2, W3, b3):
    src = edge_index[0].astype(jnp.int32)
    dst = edge_index[1].astype(jnp.int32)
    pad_e = EPAD - E
    src_p = jnp.concatenate(
        [src, jnp.zeros((pad_e,), jnp.int32)]).reshape(NW * NIG, IDXG, B)
    dst_pad_rows = N + jnp.arange(pad_e, dtype=jnp.int32) % (NPAD - N)
    dst_p = jnp.concatenate([dst, dst_pad_rows]).reshape(NW * NIG, IDXG, B)
    dst_c = dst_p.reshape(NW, CHUNKS, B)
    x_p = jnp.pad(x, ((0, NPAD - N), (0, 0)))
    zD = jnp.zeros((RPT, D), jnp.float32)
    onesD = jnp.ones((B, D), jnp.float32)
    b1r = b1.reshape(1, D)
    b2r = b2.reshape(1, D)
    b3r = b3.reshape(1, D)

    degp = _deg_kernel(dst_c, onesD, zD).reshape(2, NPAD, D)
    y1, disb = _t1(x_p, W1, degp)
    s1 = _scatter_kernel(y1, src_p, dst_c, zD).reshape(2, NPAD, D)
    y2 = _tmid(s1, y1, disb, b1r, W2)
    s2 = _scatter_kernel(y2, src_p, dst_c, zD).reshape(2, NPAD, D)
    y3 = _tmid(s2, y2, disb, b2r, W3)
    s3 = _scatter_kernel(y3, src_p, dst_c, zD).reshape(2, NPAD, D)
    out = _tfin(s3, y3, disb, b3r)
    return out[:N]


# speedup vs baseline: 1.0165x; 1.0165x over previous
import functools

import jax
import jax.numpy as jnp
from jax import lax
from jax.experimental import pallas as pl
from jax.experimental.pallas import tpu as pltpu
from jax.experimental.pallas import tpu_sc as plsc

N = 10000
D = 128
E = 320000
NPAD = 10240
NW = 32
B = 128
C0 = 54
C1 = 106
CMAX = max(C0, C1)
NCHUNK = 16 * (C0 + C1)
CHUNKS = 80
EPT = CHUNKS * B
EPAD = EPT * NW
RPT = NPAD // 16
RB = 512
GRID = NPAD // RB

_mesh = plsc.VectorSubcoreMesh(core_axis_name="c", subcore_axis_name="s")


@functools.partial(
    pl.kernel,
    mesh=_mesh,
    out_type=jax.ShapeDtypeStruct((2 * NPAD, D), jnp.float32),
    scratch_types=[
        pltpu.VMEM((CHUNKS, B), jnp.int32),
        pltpu.VMEM((B, D), jnp.float32),
        pltpu.VMEM_SHARED((NPAD, D), jnp.float32),
    ],
)
def _deg_kernel(dst_hbm, ones_hbm, zeros_hbm, out_hbm, dst_v, ones_v, acc):
    cid = lax.axis_index("c")
    sid = lax.axis_index("s")
    wid = sid * 2 + cid
    pltpu.sync_copy(zeros_hbm, acc.at[pl.ds(sid * RPT, RPT)])
    pltpu.sync_copy(ones_hbm, ones_v)
    pltpu.sync_copy(dst_hbm.at[wid], dst_v)
    plsc.subcore_barrier()

    def body(j, c):
        pltpu.sync_copy(ones_v, acc.at[dst_v.at[j]], add=True)
        return c

    lax.fori_loop(0, CHUNKS, body, 0)
    plsc.subcore_barrier()
    pltpu.sync_copy(acc.at[pl.ds(sid * RPT, RPT)],
                    out_hbm.at[pl.ds(cid * NPAD + sid * RPT, RPT)])


@functools.partial(
    pl.kernel,
    mesh=_mesh,
    out_type=jax.ShapeDtypeStruct((2 * NPAD, D), jnp.float32),
    scratch_types=[
        pltpu.VMEM((CMAX, B), jnp.int32),
        pltpu.VMEM((CMAX, B), jnp.int32),
        pltpu.VMEM((B, D), jnp.float32),
        pltpu.VMEM_SHARED((NPAD, D), jnp.float32),
        pltpu.SemaphoreType.DMA,
    ],
)
def _scatter_kernel(y_hbm, src_hbm, dst_hbm, zeros_hbm, out_hbm,
                    src_v, dst_v, rows_v, acc, sem):
    cid = lax.axis_index("c")
    sid = lax.axis_index("s")
    wid = sid * 2 + cid
    pltpu.sync_copy(zeros_hbm, acc.at[pl.ds(sid * RPT, RPT)])
    pltpu.sync_copy(src_hbm.at[wid], src_v)
    pltpu.sync_copy(dst_hbm.at[wid], dst_v)
    plsc.subcore_barrier()

    def body(j, c):
        pltpu.async_copy(y_hbm.at[src_v.at[j]], rows_v, sem).wait()
        pltpu.sync_copy(rows_v, acc.at[dst_v.at[j]], add=True)
        return c

    myc = jnp.where(cid == 0, C0, C1)
    lax.fori_loop(0, myc, body, 0)
    plsc.subcore_barrier()
    pltpu.sync_copy(acc.at[pl.ds(sid * RPT, RPT)],
                    out_hbm.at[pl.ds(cid * NPAD + sid * RPT, RPT)])


def _t1_body(x_ref, w_ref, degp_ref, y_ref, disb_ref):
    c = degp_ref[0, :, 0:1] + degp_ref[1, :, 0:1] + 1.0
    disb = lax.rsqrt(jnp.broadcast_to(c, (RB, D)))
    disb_ref[...] = disb
    y_ref[...] = jnp.dot(x_ref[...], w_ref[...],
                         preferred_element_type=jnp.float32) * disb


_t1 = pl.pallas_call(
    _t1_body,
    grid=(GRID,),
    in_specs=[
        pl.BlockSpec((RB, D), lambda i: (i, 0)),
        pl.BlockSpec((D, D), lambda i: (0, 0)),
        pl.BlockSpec((2, RB, D), lambda i: (0, i, 0)),
    ],
    out_specs=[
        pl.BlockSpec((RB, D), lambda i: (i, 0)),
        pl.BlockSpec((RB, D), lambda i: (i, 0)),
    ],
    out_shape=[
        jax.ShapeDtypeStruct((NPAD, D), jnp.float32),
        jax.ShapeDtypeStruct((NPAD, D), jnp.float32),
    ],
)


def _tmid_body(s_ref, y_ref, disb_ref, b_ref, w_ref, o_ref):
    disb = disb_ref[...]
    h = jnp.maximum((s_ref[0] + s_ref[1] + y_ref[...]) * disb + b_ref[...],
                    0.0)
    o_ref[...] = jnp.dot(h, w_ref[...],
                         preferred_element_type=jnp.float32) * disb


_tmid = pl.pallas_call(
    _tmid_body,
    grid=(GRID,),
    in_specs=[
        pl.BlockSpec((2, RB, D), lambda i: (0, i, 0)),
        pl.BlockSpec((RB, D), lambda i: (i, 0)),
        pl.BlockSpec((RB, D), lambda i: (i, 0)),
        pl.BlockSpec((1, D), lambda i: (0, 0)),
        pl.BlockSpec((D, D), lambda i: (0, 0)),
    ],
    out_specs=pl.BlockSpec((RB, D), lambda i: (i, 0)),
    out_shape=jax.ShapeDtypeStruct((NPAD, D), jnp.float32),
)


def _tfin_body(s_ref, y_ref, disb_ref, b_ref, o_ref):
    o_ref[...] = ((s_ref[0] + s_ref[1] + y_ref[...]) * disb_ref[...]
                  + b_ref[...])


_tfin = pl.pallas_call(
    _tfin_body,
    grid=(GRID,),
    in_specs=[
        pl.BlockSpec((2, RB, D), lambda i: (0, i, 0)),
        pl.BlockSpec((RB, D), lambda i: (i, 0)),
        pl.BlockSpec((RB, D), lambda i: (i, 0)),
        pl.BlockSpec((1, D), lambda i: (0, 0)),
    ],
    out_specs=pl.BlockSpec((RB, D), lambda i: (i, 0)),
    out_shape=jax.ShapeDtypeStruct((NPAD, D), jnp.float32),
)


def kernel(x, edge_index, W1, b1, W2, b2, W3, b3):
    src = edge_index[0].astype(jnp.int32)
    dst = edge_index[1].astype(jnp.int32)
    pad_e = EPAD - E
    src_flat = jnp.concatenate([src, jnp.zeros((pad_e,), jnp.int32)])
    dst_pad_rows = N + jnp.arange(pad_e, dtype=jnp.int32) % (NPAD - N)
    dst_flat = jnp.concatenate([dst, dst_pad_rows])
    dst_c = dst_flat.reshape(NW, CHUNKS, B)

    def rebalance(flat, fill):
        ch = flat.reshape(NCHUNK, B)
        p0 = ch[:16 * C0].reshape(16, C0, B)
        p0 = jnp.concatenate(
            [p0, jnp.full((16, CMAX - C0, B), fill, jnp.int32)], axis=1)
        p1 = ch[16 * C0:].reshape(16, C1, B)
        p1 = jnp.concatenate(
            [p1, jnp.full((16, CMAX - C1, B), fill, jnp.int32)], axis=1)
        return jnp.stack([p0, p1], axis=1).reshape(NW, CMAX, B)

    src_p = rebalance(src_flat, 0)
    dst_p = rebalance(dst_flat, NPAD - 1)
    x_p = jnp.pad(x, ((0, NPAD - N), (0, 0)))
    zD = jnp.zeros((RPT, D), jnp.float32)
    onesD = jnp.ones((B, D), jnp.float32)
    b1r = b1.reshape(1, D)
    b2r = b2.reshape(1, D)
    b3r = b3.reshape(1, D)

    degp = _deg_kernel(dst_c, onesD, zD).reshape(2, NPAD, D)
    y1, disb = _t1(x_p, W1, degp)
    s1 = _scatter_kernel(y1, src_p, dst_p, zD).reshape(2, NPAD, D)
    y2 = _tmid(s1, y1, disb, b1r, W2)
    s2 = _scatter_kernel(y2, src_p, dst_p, zD).reshape(2, NPAD, D)
    y3 = _tmid(s2, y2, disb, b2r, W3)
    s3 = _scatter_kernel(y3, src_p, dst_p, zD).reshape(2, NPAD, D)
    out = _tfin(s3, y3, disb, b3r)
    return out[:N]


# speedup vs baseline: 1.0901x; 1.0724x over previous
import functools

import jax
import jax.numpy as jnp
from jax import lax
from jax.experimental import pallas as pl
from jax.experimental.pallas import tpu as pltpu
from jax.experimental.pallas import tpu_sc as plsc

N = 10000
D = 128
E = 320000
NPAD = 10240
NW = 32
B = 128
C0 = 106
C1 = 54
CMAX = max(C0, C1)
NCHUNK = 16 * (C0 + C1)
CHUNKS = 80
EPT = CHUNKS * B
EPAD = EPT * NW
RPT = NPAD // 16
RB = 512
GRID = NPAD // RB

_mesh = plsc.VectorSubcoreMesh(core_axis_name="c", subcore_axis_name="s")


@functools.partial(
    pl.kernel,
    mesh=_mesh,
    out_type=jax.ShapeDtypeStruct((2 * NPAD, D), jnp.float32),
    scratch_types=[
        pltpu.VMEM((CHUNKS, B), jnp.int32),
        pltpu.VMEM((B, D), jnp.float32),
        pltpu.VMEM_SHARED((NPAD, D), jnp.float32),
    ],
)
def _deg_kernel(dst_hbm, ones_hbm, zeros_hbm, out_hbm, dst_v, ones_v, acc):
    cid = lax.axis_index("c")
    sid = lax.axis_index("s")
    wid = sid * 2 + cid
    pltpu.sync_copy(zeros_hbm, acc.at[pl.ds(sid * RPT, RPT)])
    pltpu.sync_copy(ones_hbm, ones_v)
    pltpu.sync_copy(dst_hbm.at[wid], dst_v)
    plsc.subcore_barrier()

    def body(j, c):
        pltpu.sync_copy(ones_v, acc.at[dst_v.at[j]], add=True)
        return c

    lax.fori_loop(0, CHUNKS, body, 0)
    plsc.subcore_barrier()
    pltpu.sync_copy(acc.at[pl.ds(sid * RPT, RPT)],
                    out_hbm.at[pl.ds(cid * NPAD + sid * RPT, RPT)])


@functools.partial(
    pl.kernel,
    mesh=_mesh,
    out_type=jax.ShapeDtypeStruct((2 * NPAD, D), jnp.float32),
    scratch_types=[
        pltpu.VMEM((CMAX, B), jnp.int32),
        pltpu.VMEM((CMAX, B), jnp.int32),
        pltpu.VMEM((B, D), jnp.float32),
        pltpu.VMEM_SHARED((NPAD, D), jnp.float32),
        pltpu.SemaphoreType.DMA,
    ],
)
def _scatter_kernel(y_hbm, src_hbm, dst_hbm, zeros_hbm, out_hbm,
                    src_v, dst_v, rows_v, acc, sem):
    cid = lax.axis_index("c")
    sid = lax.axis_index("s")
    wid = sid * 2 + cid
    pltpu.sync_copy(zeros_hbm, acc.at[pl.ds(sid * RPT, RPT)])
    pltpu.sync_copy(src_hbm.at[wid], src_v)
    pltpu.sync_copy(dst_hbm.at[wid], dst_v)
    plsc.subcore_barrier()

    def body(j, c):
        pltpu.async_copy(y_hbm.at[src_v.at[j]], rows_v, sem).wait()
        pltpu.sync_copy(rows_v, acc.at[dst_v.at[j]], add=True)
        return c

    myc = jnp.where(cid == 0, C0, C1)
    lax.fori_loop(0, myc, body, 0)
    plsc.subcore_barrier()
    pltpu.sync_copy(acc.at[pl.ds(sid * RPT, RPT)],
                    out_hbm.at[pl.ds(cid * NPAD + sid * RPT, RPT)])


def _t1_body(x_ref, w_ref, degp_ref, y_ref, disb_ref):
    c = degp_ref[0, :, 0:1] + degp_ref[1, :, 0:1] + 1.0
    disb = lax.rsqrt(jnp.broadcast_to(c, (RB, D)))
    disb_ref[...] = disb
    y_ref[...] = jnp.dot(x_ref[...], w_ref[...],
                         preferred_element_type=jnp.float32) * disb


_t1 = pl.pallas_call(
    _t1_body,
    grid=(GRID,),
    in_specs=[
        pl.BlockSpec((RB, D), lambda i: (i, 0)),
        pl.BlockSpec((D, D), lambda i: (0, 0)),
        pl.BlockSpec((2, RB, D), lambda i: (0, i, 0)),
    ],
    out_specs=[
        pl.BlockSpec((RB, D), lambda i: (i, 0)),
        pl.BlockSpec((RB, D), lambda i: (i, 0)),
    ],
    out_shape=[
        jax.ShapeDtypeStruct((NPAD, D), jnp.float32),
        jax.ShapeDtypeStruct((NPAD, D), jnp.float32),
    ],
)


def _tmid_body(s_ref, y_ref, disb_ref, b_ref, w_ref, o_ref):
    disb = disb_ref[...]
    h = jnp.maximum((s_ref[0] + s_ref[1] + y_ref[...]) * disb + b_ref[...],
                    0.0)
    o_ref[...] = jnp.dot(h, w_ref[...],
                         preferred_element_type=jnp.float32) * disb


_tmid = pl.pallas_call(
    _tmid_body,
    grid=(GRID,),
    in_specs=[
        pl.BlockSpec((2, RB, D), lambda i: (0, i, 0)),
        pl.BlockSpec((RB, D), lambda i: (i, 0)),
        pl.BlockSpec((RB, D), lambda i: (i, 0)),
        pl.BlockSpec((1, D), lambda i: (0, 0)),
        pl.BlockSpec((D, D), lambda i: (0, 0)),
    ],
    out_specs=pl.BlockSpec((RB, D), lambda i: (i, 0)),
    out_shape=jax.ShapeDtypeStruct((NPAD, D), jnp.float32),
)


def _tfin_body(s_ref, y_ref, disb_ref, b_ref, o_ref):
    o_ref[...] = ((s_ref[0] + s_ref[1] + y_ref[...]) * disb_ref[...]
                  + b_ref[...])


_tfin = pl.pallas_call(
    _tfin_body,
    grid=(GRID,),
    in_specs=[
        pl.BlockSpec((2, RB, D), lambda i: (0, i, 0)),
        pl.BlockSpec((RB, D), lambda i: (i, 0)),
        pl.BlockSpec((RB, D), lambda i: (i, 0)),
        pl.BlockSpec((1, D), lambda i: (0, 0)),
    ],
    out_specs=pl.BlockSpec((RB, D), lambda i: (i, 0)),
    out_shape=jax.ShapeDtypeStruct((NPAD, D), jnp.float32),
)


def kernel(x, edge_index, W1, b1, W2, b2, W3, b3):
    src = edge_index[0].astype(jnp.int32)
    dst = edge_index[1].astype(jnp.int32)
    pad_e = EPAD - E
    src_flat = jnp.concatenate([src, jnp.zeros((pad_e,), jnp.int32)])
    dst_pad_rows = N + jnp.arange(pad_e, dtype=jnp.int32) % (NPAD - N)
    dst_flat = jnp.concatenate([dst, dst_pad_rows])
    dst_c = dst_flat.reshape(NW, CHUNKS, B)

    def rebalance(flat, fill):
        ch = flat.reshape(NCHUNK, B)
        p0 = ch[:16 * C0].reshape(16, C0, B)
        p0 = jnp.concatenate(
            [p0, jnp.full((16, CMAX - C0, B), fill, jnp.int32)], axis=1)
        p1 = ch[16 * C0:].reshape(16, C1, B)
        p1 = jnp.concatenate(
            [p1, jnp.full((16, CMAX - C1, B), fill, jnp.int32)], axis=1)
        return jnp.stack([p0, p1], axis=1).reshape(NW, CMAX, B)

    src_p = rebalance(src_flat, 0)
    dst_p = rebalance(dst_flat, NPAD - 1)
    x_p = jnp.pad(x, ((0, NPAD - N), (0, 0)))
    zD = jnp.zeros((RPT, D), jnp.float32)
    onesD = jnp.ones((B, D), jnp.float32)
    b1r = b1.reshape(1, D)
    b2r = b2.reshape(1, D)
    b3r = b3.reshape(1, D)

    degp = _deg_kernel(dst_c, onesD, zD).reshape(2, NPAD, D)
    y1, disb = _t1(x_p, W1, degp)
    s1 = _scatter_kernel(y1, src_p, dst_p, zD).reshape(2, NPAD, D)
    y2 = _tmid(s1, y1, disb, b1r, W2)
    s2 = _scatter_kernel(y2, src_p, dst_p, zD).reshape(2, NPAD, D)
    y3 = _tmid(s2, y2, disb, b2r, W3)
    s3 = _scatter_kernel(y3, src_p, dst_p, zD).reshape(2, NPAD, D)
    out = _tfin(s3, y3, disb, b3r)
    return out[:N]


# speedup vs baseline: 1.5277x; 1.4014x over previous
import functools

import jax
import jax.numpy as jnp
from jax import lax
from jax.experimental import pallas as pl
from jax.experimental.pallas import tpu as pltpu
from jax.experimental.pallas import tpu_sc as plsc

N = 10000
D = 128
E = 320000
NPAD = 10240
NW = 32
B = 128
CHUNKS = 79
EPT = CHUNKS * B
EPAD = EPT * NW
RPT = NPAD // 16
RB = 512
GRID = NPAD // RB

_mesh = plsc.VectorSubcoreMesh(core_axis_name="c", subcore_axis_name="s")


@functools.partial(
    pl.kernel,
    mesh=_mesh,
    out_type=jax.ShapeDtypeStruct((2 * NPAD, D), jnp.float32),
    scratch_types=[
        pltpu.VMEM((CHUNKS, B), jnp.int32),
        pltpu.VMEM((B, D), jnp.float32),
        pltpu.VMEM_SHARED((NPAD, D), jnp.float32),
    ],
)
def _deg_kernel(dst_hbm, ones_hbm, zeros_hbm, out_hbm, dst_v, ones_v, acc):
    cid = lax.axis_index("c")
    sid = lax.axis_index("s")
    wid = sid * 2 + cid
    pltpu.sync_copy(zeros_hbm, acc.at[pl.ds(sid * RPT, RPT)])
    pltpu.sync_copy(ones_hbm, ones_v)
    pltpu.sync_copy(dst_hbm.at[wid], dst_v)
    plsc.subcore_barrier()

    def body(j, c):
        pltpu.sync_copy(ones_v, acc.at[dst_v.at[j]], add=True)
        return c

    lax.fori_loop(0, CHUNKS, body, 0)
    plsc.subcore_barrier()
    pltpu.sync_copy(acc.at[pl.ds(sid * RPT, RPT)],
                    out_hbm.at[pl.ds(cid * NPAD + sid * RPT, RPT)])


@functools.partial(
    pl.kernel,
    mesh=_mesh,
    out_type=jax.ShapeDtypeStruct((2 * NPAD, D), jnp.float32),
    scratch_types=[
        pltpu.VMEM((CHUNKS, B), jnp.int32),
        pltpu.VMEM((CHUNKS, B), jnp.int32),
        pltpu.VMEM((B, D), jnp.float32),
        pltpu.VMEM_SHARED((NPAD, D), jnp.float32),
        pltpu.SemaphoreType.DMA,
    ],
)
def _scatter_kernel(y_hbm, src_hbm, dst_hbm, zeros_hbm, out_hbm,
                    src_v, dst_v, rows_v, acc, sem):
    cid = lax.axis_index("c")
    sid = lax.axis_index("s")
    wid = sid * 2 + cid
    pltpu.sync_copy(zeros_hbm, acc.at[pl.ds(sid * RPT, RPT)])
    pltpu.sync_copy(src_hbm.at[wid], src_v)
    pltpu.sync_copy(dst_hbm.at[wid], dst_v)
    plsc.subcore_barrier()

    def body(j, c):
        pltpu.async_copy(y_hbm.at[src_v.at[j]], rows_v, sem).wait()
        pltpu.sync_copy(rows_v, acc.at[dst_v.at[j]], add=True)
        return c

    lax.fori_loop(0, CHUNKS, body, 0)
    plsc.subcore_barrier()
    pltpu.sync_copy(acc.at[pl.ds(sid * RPT, RPT)],
                    out_hbm.at[pl.ds(cid * NPAD + sid * RPT, RPT)])


def _t1_body(x_ref, w_ref, degp_ref, y_ref, disb_ref):
    c = degp_ref[0, :, 0:1] + degp_ref[1, :, 0:1] + 1.0
    disb = lax.rsqrt(jnp.broadcast_to(c, (RB, D)))
    disb_ref[...] = disb
    y_ref[...] = jnp.dot(x_ref[...], w_ref[...],
                         preferred_element_type=jnp.float32) * disb


_t1 = pl.pallas_call(
    _t1_body,
    grid=(GRID,),
    in_specs=[
        pl.BlockSpec((RB, D), lambda i: (i, 0)),
        pl.BlockSpec((D, D), lambda i: (0, 0)),
        pl.BlockSpec((2, RB, D), lambda i: (0, i, 0)),
    ],
    out_specs=[
        pl.BlockSpec((RB, D), lambda i: (i, 0)),
        pl.BlockSpec((RB, D), lambda i: (i, 0)),
    ],
    out_shape=[
        jax.ShapeDtypeStruct((NPAD, D), jnp.float32),
        jax.ShapeDtypeStruct((NPAD, D), jnp.float32),
    ],
)


def _tmid_body(s_ref, y_ref, disb_ref, b_ref, w_ref, o_ref):
    disb = disb_ref[...]
    h = jnp.maximum((s_ref[0] + s_ref[1] + y_ref[...]) * disb + b_ref[...],
                    0.0)
    o_ref[...] = jnp.dot(h, w_ref[...],
                         preferred_element_type=jnp.float32) * disb


_tmid = pl.pallas_call(
    _tmid_body,
    grid=(GRID,),
    in_specs=[
        pl.BlockSpec((2, RB, D), lambda i: (0, i, 0)),
        pl.BlockSpec((RB, D), lambda i: (i, 0)),
        pl.BlockSpec((RB, D), lambda i: (i, 0)),
        pl.BlockSpec((1, D), lambda i: (0, 0)),
        pl.BlockSpec((D, D), lambda i: (0, 0)),
    ],
    out_specs=pl.BlockSpec((RB, D), lambda i: (i, 0)),
    out_shape=jax.ShapeDtypeStruct((NPAD, D), jnp.float32),
)


def _tfin_body(s_ref, y_ref, disb_ref, b_ref, o_ref):
    o_ref[...] = ((s_ref[0] + s_ref[1] + y_ref[...]) * disb_ref[...]
                  + b_ref[...])


_tfin = pl.pallas_call(
    _tfin_body,
    grid=(GRID,),
    in_specs=[
        pl.BlockSpec((2, RB, D), lambda i: (0, i, 0)),
        pl.BlockSpec((RB, D), lambda i: (i, 0)),
        pl.BlockSpec((RB, D), lambda i: (i, 0)),
        pl.BlockSpec((1, D), lambda i: (0, 0)),
    ],
    out_specs=pl.BlockSpec((RB, D), lambda i: (i, 0)),
    out_shape=jax.ShapeDtypeStruct((NPAD, D), jnp.float32),
)


def kernel(x, edge_index, W1, b1, W2, b2, W3, b3):
    src = edge_index[0].astype(jnp.int32)
    dst = edge_index[1].astype(jnp.int32)
    pad_e = EPAD - E
    src_p = jnp.concatenate(
        [src, jnp.zeros((pad_e,), jnp.int32)]).reshape(NW, CHUNKS, B)
    dst_p = jnp.concatenate(
        [dst, jnp.full((pad_e,), NPAD - 1, jnp.int32)]).reshape(NW, CHUNKS, B)
    x_p = jnp.pad(x, ((0, NPAD - N), (0, 0)))
    zD = jnp.zeros((RPT, D), jnp.float32)
    onesD = jnp.ones((B, D), jnp.float32)
    b1r = b1.reshape(1, D)
    b2r = b2.reshape(1, D)
    b3r = b3.reshape(1, D)

    degp = _deg_kernel(dst_p, onesD, zD).reshape(2, NPAD, D)
    y1, disb = _t1(x_p, W1, degp)
    s1 = _scatter_kernel(y1, src_p, dst_p, zD).reshape(2, NPAD, D)
    y2 = _tmid(s1, y1, disb, b1r, W2)
    s2 = _scatter_kernel(y2, src_p, dst_p, zD).reshape(2, NPAD, D)
    y3 = _tmid(s2, y2, disb, b2r, W3)
    s3 = _scatter_kernel(y3, src_p, dst_p, zD).reshape(2, NPAD, D)
    out = _tfin(s3, y3, disb, b3r)
    return out[:N]
